# Initial kernel scaffold; baseline (speedup 1.0000x reference)
import jax
import jax.numpy as jnp
from jax.experimental import pallas as pl


def kernel(z, weight):
    raise NotImplementedError("write your pallas kernel here")


# speedup vs baseline: 103.5090x; 103.5090x over previous
import functools

import jax
import jax.numpy as jnp
from jax import lax
from jax.experimental import pallas as pl
from jax.experimental.pallas import tpu as pltpu
from jax.experimental.pallas import tpu_sc as plsc

NUM_EMBED = 8192
EMBED_DIM = 256
BETA = 0.25
TOK = 16384
TBLK = 256
GRID = TOK // TBLK


def _top1_body(z_ref, w_ref, idx_ref, perp_ref, counts_ref):
    step = pl.program_id(0)
    z = z_ref[...]
    w = w_ref[...]
    zn = z / jnp.maximum(jnp.sqrt(jnp.sum(z * z, axis=1, keepdims=True)), 1e-12)
    wn = w / jnp.maximum(jnp.sqrt(jnp.sum(w * w, axis=1, keepdims=True)), 1e-12)
    d = lax.dot_general(zn, wn, (((1,), (1,)), ((), ())),
                        preferred_element_type=jnp.float32)
    m = jnp.max(d, axis=1, keepdims=True)
    col = lax.broadcasted_iota(jnp.int32, d.shape, 1)
    idx = jnp.max(jnp.where(d >= m, col, -1), axis=1)
    idx_ref[0, 0, :] = idx

    @pl.when(step == 0)
    def _init():
        counts_ref[...] = jnp.zeros_like(counts_ref)

    onehot = (col == idx[:, None]).astype(jnp.float32)
    counts_ref[...] += jnp.sum(onehot, axis=0)[None, :]

    @pl.when(step == GRID - 1)
    def _finish():
        p = counts_ref[0, :] * (1.0 / TOK)
        ent = -jnp.sum(p * jnp.log(p + 1e-10))
        perp_ref[0, 0] = jnp.exp(ent)


def _top1(z_flat, weight):
    idx3, perp = pl.pallas_call(
        _top1_body,
        grid=(GRID,),
        in_specs=[
            pl.BlockSpec((TBLK, EMBED_DIM), lambda i: (i, 0)),
            pl.BlockSpec((NUM_EMBED, EMBED_DIM), lambda i: (0, 0)),
        ],
        out_specs=[
            pl.BlockSpec((1, 1, TBLK), lambda i: (i, 0, 0)),
            pl.BlockSpec(memory_space=pltpu.SMEM),
        ],
        out_shape=[
            jax.ShapeDtypeStruct((GRID, 1, TBLK), jnp.int32),
            jax.ShapeDtypeStruct((1, 1), jnp.float32),
        ],
        scratch_shapes=[pltpu.VMEM((1, NUM_EMBED), jnp.float32)],
    )(z_flat, weight)
    return idx3.reshape(TOK), perp[0, 0]


def _gather_rows(weight, idx):
    info = plsc.get_sparse_core_info()
    NW = info.num_cores * info.num_subcores
    b_per_w = TOK // NW
    n_chunks = b_per_w // 128
    mesh = plsc.VectorSubcoreMesh(core_axis_name="c", subcore_axis_name="s")

    @functools.partial(
        pl.kernel, mesh=mesh,
        out_type=jax.ShapeDtypeStruct((TOK, EMBED_DIM), jnp.float32),
        scratch_types=[
            pltpu.VMEM((n_chunks, 128), jnp.int32),
            pltpu.VMEM((128, EMBED_DIM), jnp.float32),
            pltpu.SemaphoreType.DMA,
        ],
    )
    def k(w_hbm, idx_hbm, out_hbm, idx_v, rows_v, sem):
        wid = lax.axis_index("s") * info.num_cores + lax.axis_index("c")
        base = wid * b_per_w
        pltpu.sync_copy(idx_hbm.at[pl.ds(wid * n_chunks, n_chunks)], idx_v)
        for j in range(n_chunks):
            pltpu.async_copy(w_hbm.at[idx_v.at[j]], rows_v, sem).wait()
            pltpu.sync_copy(rows_v, out_hbm.at[pl.ds(base + j * 128, 128)])

    return k(weight, idx.reshape(TOK // 128, 128))


def _loss_body(zq_ref, z_ref, st_ref, loss_ref, acc_ref):
    step = pl.program_id(0)

    @pl.when(step == 0)
    def _init():
        acc_ref[0] = 0.0

    z = z_ref[...]
    diff = zq_ref[...] - z
    st_ref[...] = z + diff
    acc_ref[0] += jnp.sum(diff * diff)

    @pl.when(step == pl.num_programs(0) - 1)
    def _finish():
        loss_ref[0, 0] = acc_ref[0] * ((1.0 + BETA) / (TOK * EMBED_DIM))


def _loss(z_q, z_flat):
    blk = 2048
    st, out = pl.pallas_call(
        _loss_body,
        grid=(TOK // blk,),
        in_specs=[
            pl.BlockSpec((blk, EMBED_DIM), lambda i: (i, 0)),
            pl.BlockSpec((blk, EMBED_DIM), lambda i: (i, 0)),
        ],
        out_specs=[
            pl.BlockSpec((blk, EMBED_DIM), lambda i: (i, 0)),
            pl.BlockSpec(memory_space=pltpu.SMEM),
        ],
        out_shape=[
            jax.ShapeDtypeStruct((TOK, EMBED_DIM), jnp.float32),
            jax.ShapeDtypeStruct((1, 1), jnp.float32),
        ],
        scratch_shapes=[pltpu.SMEM((1,), jnp.float32)],
    )(z_q, z_flat)
    return st, out[0, 0]


def kernel(z, weight):
    b, c, h, w = z.shape
    z_flat = jnp.transpose(z, (0, 2, 3, 1)).reshape(TOK, EMBED_DIM)
    encoding_indices, perplexity = _top1(z_flat, weight)
    z_q = _gather_rows(weight, encoding_indices)
    z_q_st, loss = _loss(z_q, z_flat)
    z_q_out = jnp.transpose(z_q_st.reshape(b, h, w, c), (0, 3, 1, 2))
    return z_q_out, loss, perplexity, encoding_indices


# speedup vs baseline: 168.9799x; 1.6325x over previous
import functools

import jax
import jax.numpy as jnp
from jax import lax
from jax.experimental import pallas as pl
from jax.experimental.pallas import tpu as pltpu
from jax.experimental.pallas import tpu_sc as plsc

NUM_EMBED = 8192
EMBED_DIM = 256
BETA = 0.25
TOK = 16384
TBLK = 256
GRID = TOK // TBLK
NCHUNK = 8
CW = NUM_EMBED // NCHUNK
LBLK = 2048


def _wnorm_body(w_ref, wn_ref):
    w = w_ref[...]
    wn_ref[...] = w / jnp.maximum(
        jnp.sqrt(jnp.sum(w * w, axis=1, keepdims=True)), 1e-12)


def _wnorm(weight):
    return pl.pallas_call(
        _wnorm_body,
        out_shape=jax.ShapeDtypeStruct((NUM_EMBED, EMBED_DIM), jnp.float32),
    )(weight)


def _top1_body(z_ref, wn_ref, idx_ref):
    z = z_ref[...]
    zn = z / jnp.maximum(
        jnp.sqrt(jnp.sum(z * z, axis=1, keepdims=True)), 1e-12)
    m = jnp.full((TBLK,), -jnp.inf, jnp.float32)
    idx = jnp.zeros((TBLK,), jnp.int32)
    for c in range(NCHUNK):
        wc = wn_ref[pl.ds(c * CW, CW), :]
        dc = lax.dot_general(zn, wc, (((1,), (1,)), ((), ())),
                             preferred_element_type=jnp.float32)
        mc = jnp.max(dc, axis=1)
        colc = lax.broadcasted_iota(jnp.int32, dc.shape, 1) + c * CW
        idxc = jnp.max(jnp.where(dc >= mc[:, None], colc, -1), axis=1)
        take = mc >= m
        idx = jnp.where(take, idxc, idx)
        m = jnp.where(take, mc, m)
    idx_ref[0, 0, :] = idx


def _top1(z_flat, wn):
    idx3 = pl.pallas_call(
        _top1_body,
        grid=(GRID,),
        in_specs=[
            pl.BlockSpec((TBLK, EMBED_DIM), lambda i: (i, 0)),
            pl.BlockSpec((NUM_EMBED, EMBED_DIM), lambda i: (0, 0)),
        ],
        out_specs=pl.BlockSpec((1, 1, TBLK), lambda i: (i, 0, 0)),
        out_shape=jax.ShapeDtypeStruct((GRID, 1, TBLK), jnp.int32),
    )(z_flat, wn)
    return idx3.reshape(TOK)


def _gather_rows(weight, idx):
    info = plsc.get_sparse_core_info()
    NW = info.num_cores * info.num_subcores
    b_per_w = TOK // NW
    n_chunks = b_per_w // 128
    mesh = plsc.VectorSubcoreMesh(core_axis_name="c", subcore_axis_name="s")

    @functools.partial(
        pl.kernel, mesh=mesh,
        out_type=jax.ShapeDtypeStruct((TOK, EMBED_DIM), jnp.float32),
        scratch_types=[
            pltpu.VMEM((n_chunks, 128), jnp.int32),
            pltpu.VMEM((128, EMBED_DIM), jnp.float32),
            pltpu.SemaphoreType.DMA,
        ],
    )
    def k(w_hbm, idx_hbm, out_hbm, idx_v, rows_v, sem):
        wid = lax.axis_index("s") * info.num_cores + lax.axis_index("c")
        base = wid * b_per_w
        pltpu.sync_copy(idx_hbm.at[pl.ds(wid * n_chunks, n_chunks)], idx_v)
        for j in range(n_chunks):
            pltpu.async_copy(w_hbm.at[idx_v.at[j]], rows_v, sem).wait()
            pltpu.sync_copy(rows_v, out_hbm.at[pl.ds(base + j * 128, 128)])

    return k(weight, idx.reshape(TOK // 128, 128))


def _stats_body(zq_ref, z_ref, idx_ref, st_ref, loss_ref, perp_ref,
                acc_ref, c_ref):
    step = pl.program_id(0)

    @pl.when(step == 0)
    def _init():
        acc_ref[0] = 0.0
        c_ref[...] = jnp.zeros_like(c_ref)

    z = z_ref[...]
    diff = zq_ref[...] - z
    st_ref[...] = z + diff
    acc_ref[0] += jnp.sum(diff * diff)

    idx = idx_ref[0, 0, :]
    hi = idx >> 7
    lo = idx & 127
    hh = (lax.broadcasted_iota(jnp.int32, (64, LBLK), 0)
          == hi[None, :]).astype(jnp.float32)
    hl = (lo[:, None]
          == lax.broadcasted_iota(jnp.int32, (LBLK, 128), 1)).astype(jnp.float32)
    c_ref[...] += lax.dot_general(hh, hl, (((1,), (0,)), ((), ())),
                                  preferred_element_type=jnp.float32)

    @pl.when(step == pl.num_programs(0) - 1)
    def _finish():
        loss_ref[0, 0] = acc_ref[0] * ((1.0 + BETA) / (TOK * EMBED_DIM))
        p = c_ref[...] * (1.0 / TOK)
        ent = -jnp.sum(p * jnp.log(p + 1e-10))
        perp_ref[0, 0] = jnp.exp(ent)


def _stats(z_q, z_flat, idx):
    st, loss, perp = pl.pallas_call(
        _stats_body,
        grid=(TOK // LBLK,),
        in_specs=[
            pl.BlockSpec((LBLK, EMBED_DIM), lambda i: (i, 0)),
            pl.BlockSpec((LBLK, EMBED_DIM), lambda i: (i, 0)),
            pl.BlockSpec((1, 1, LBLK), lambda i: (i, 0, 0)),
        ],
        out_specs=[
            pl.BlockSpec((LBLK, EMBED_DIM), lambda i: (i, 0)),
            pl.BlockSpec(memory_space=pltpu.SMEM),
            pl.BlockSpec(memory_space=pltpu.SMEM),
        ],
        out_shape=[
            jax.ShapeDtypeStruct((TOK, EMBED_DIM), jnp.float32),
            jax.ShapeDtypeStruct((1, 1), jnp.float32),
            jax.ShapeDtypeStruct((1, 1), jnp.float32),
        ],
        scratch_shapes=[
            pltpu.SMEM((1,), jnp.float32),
            pltpu.VMEM((64, 128), jnp.float32),
        ],
    )(z_q, z_flat, idx.reshape(TOK // LBLK, 1, LBLK))
    return st, loss[0, 0], perp[0, 0]


def kernel(z, weight):
    b, c, h, w = z.shape
    z_flat = jnp.transpose(z, (0, 2, 3, 1)).reshape(TOK, EMBED_DIM)
    wn = _wnorm(weight)
    encoding_indices = _top1(z_flat, wn)
    z_q = _gather_rows(weight, encoding_indices)
    z_q_st, loss, perplexity = _stats(z_q, z_flat, encoding_indices)
    z_q_out = jnp.transpose(z_q_st.reshape(b, h, w, c), (0, 3, 1, 2))
    return z_q_out, loss, perplexity, encoding_indices
